# Initial kernel scaffold; baseline (speedup 1.0000x reference)
import jax
import jax.numpy as jnp
from jax.experimental import pallas as pl


def kernel(input_img, ref_img, batchsize, gray_rate):
    raise NotImplementedError("write your pallas kernel here")


# speedup vs baseline: 222.3596x; 222.3596x over previous
import functools

import jax
import jax.numpy as jnp
import numpy as np
from jax.experimental import pallas as pl
from jax.experimental.pallas import tpu as pltpu

_C = np.float32(256.0 / 255.0)

_ROWS = 24576
_COLS = 1024
_BLOCK_ROWS = 1024
_GRID = _ROWS // _BLOCK_ROWS


def _body(a_ref, b_ref, s_ref, ca_ref, cb_ref):
    i = pl.program_id(0)

    @pl.when(i == 0)
    def _init():
        s_ref[0, 0] = jnp.float32(0.0)
        ca_ref[0, 0] = jnp.int32(0)
        cb_ref[0, 0] = jnp.int32(0)

    a = a_ref[...]
    b = b_ref[...]
    d = a - b
    s_ref[0, 0] += jnp.sum(d * d)
    ca_ref[0, 0] += jnp.sum(((a >= 0.0) & (a * _C < 1.0)).astype(jnp.int32))
    cb_ref[0, 0] += jnp.sum(((b >= 0.0) & (b * _C < 1.0)).astype(jnp.int32))


@functools.partial(jax.jit, static_argnames=("interpret",))
def _reduce_pass(a, b, interpret=False):
    a2 = a.reshape(_ROWS, _COLS)
    b2 = b.reshape(_ROWS, _COLS)
    smem_out = pl.BlockSpec(memory_space=pltpu.SMEM)
    return pl.pallas_call(
        _body,
        grid=(_GRID,),
        in_specs=[
            pl.BlockSpec((_BLOCK_ROWS, _COLS), lambda i: (i, 0)),
            pl.BlockSpec((_BLOCK_ROWS, _COLS), lambda i: (i, 0)),
        ],
        out_specs=[smem_out, smem_out, smem_out],
        out_shape=[
            jax.ShapeDtypeStruct((1, 1), jnp.float32),
            jax.ShapeDtypeStruct((1, 1), jnp.int32),
            jax.ShapeDtypeStruct((1, 1), jnp.int32),
        ],
        interpret=interpret,
    )(a2, b2)


def kernel(input_img, ref_img, batchsize, gray_rate, interpret=False):
    n = input_img.size
    s, c_inp, c_ref = _reduce_pass(input_img, ref_img, interpret=interpret)
    mse = s[0, 0] / jnp.float32(n)
    dcount = (c_ref[0, 0] - c_inp[0, 0]).astype(jnp.float32)
    bsz = jnp.asarray(batchsize, jnp.float32)
    loss_intensity = (bsz * dcount) ** 2
    return mse + jnp.asarray(gray_rate, jnp.float32) * loss_intensity
